# baseline (speedup 1.0000x reference)
import functools

import jax
import jax.numpy as jnp
from jax.experimental import pallas as pl
from jax.experimental.pallas import tpu as pltpu
from jax.experimental.pallas import tpu_sc as plsc

RATIO_THRESH = 0.95
DIST_THRESH = 1.0
NEG = -3.0


def _tc_body(BN, M, NI, d0_ref, d1_ref, sim_ref, n0_ref, v1r_ref, v2r_ref,
             n1_ref, v1c_ref, v2c_ref, d1n_ref, v1s, i1s, v2s):
    i = pl.program_id(1)

    @pl.when(i == 0)
    def _():
        d1 = d1_ref[0]
        nrm = jnp.sqrt(jnp.sum(d1 * d1, axis=1, keepdims=True))
        d1n_ref[...] = d1 / jnp.maximum(nrm, 1e-12)

    d0 = d0_ref[0]
    nrm0 = jnp.sqrt(jnp.sum(d0 * d0, axis=1, keepdims=True))
    d0n = d0 / jnp.maximum(nrm0, 1e-12)
    sim = jax.lax.dot_general(
        d0n, d1n_ref[...],
        dimension_numbers=(((1,), (1,)), ((), ())),
        preferred_element_type=jnp.float32)
    sim_ref[0] = sim

    colf = jax.lax.broadcasted_iota(jnp.int32, (M, 1), 0).astype(jnp.float32)
    v1 = jnp.max(sim, axis=1, keepdims=True)
    eq = sim == v1
    eqf = eq.astype(jnp.float32)
    i1 = jax.lax.dot_general(
        eqf, colf, dimension_numbers=(((1,), (0,)), ((), ())),
        preferred_element_type=jnp.float32).astype(jnp.int32)
    v2 = jnp.max(jnp.where(eq, NEG, sim), axis=1, keepdims=True)
    n0_ref[0, 0] = i1
    v1r_ref[0, 0] = v1
    v2r_ref[0, 0] = v2

    rowf = (jax.lax.broadcasted_iota(jnp.int32, (1, BN), 1) + i * BN
            ).astype(jnp.float32)
    c1 = jnp.max(sim, axis=0, keepdims=True)
    eqc = sim == c1
    eqcf = eqc.astype(jnp.float32)
    gi = jax.lax.dot_general(
        rowf, eqcf, dimension_numbers=(((1,), (0,)), ((), ())),
        preferred_element_type=jnp.float32).astype(jnp.int32)
    c2 = jnp.max(jnp.where(eqc, NEG, sim), axis=0, keepdims=True)

    @pl.when(i == 0)
    def _():
        v1s[...] = c1
        i1s[...] = gi
        v2s[...] = c2

    @pl.when(i > 0)
    def _():
        pv1, pi1, pv2 = v1s[...], i1s[...], v2s[...]
        keep = pv1 >= c1
        v2s[...] = jnp.where(keep, jnp.maximum(pv2, c1), jnp.maximum(pv1, c2))
        i1s[...] = jnp.where(keep, pi1, gi)
        v1s[...] = jnp.maximum(pv1, c1)

    @pl.when(i == NI - 1)
    def _():
        n1_ref[0] = i1s[...]
        v1c_ref[0] = v1s[...]
        v2c_ref[0] = v2s[...]


def _tc_matmul_top2(desc0, desc1, BN=128, interpret=False):
    B, N, C = desc0.shape
    M = desc1.shape[1]
    NI = N // BN
    f32, i32 = jnp.float32, jnp.int32
    out_shape = [
        jax.ShapeDtypeStruct((B, N, M), f32),
        jax.ShapeDtypeStruct((B, NI, BN, 1), i32),
        jax.ShapeDtypeStruct((B, NI, BN, 1), f32),
        jax.ShapeDtypeStruct((B, NI, BN, 1), f32),
        jax.ShapeDtypeStruct((B, 1, M), i32),
        jax.ShapeDtypeStruct((B, 1, M), f32),
        jax.ShapeDtypeStruct((B, 1, M), f32),
    ]
    grid = (B, NI)
    in_specs = [
        pl.BlockSpec((1, BN, C), lambda b, i: (b, i, 0)),
        pl.BlockSpec((1, M, C), lambda b, i: (b, 0, 0)),
    ]
    out_specs = [
        pl.BlockSpec((1, BN, M), lambda b, i: (b, i, 0)),
        pl.BlockSpec((1, 1, BN, 1), lambda b, i: (b, i, 0, 0)),
        pl.BlockSpec((1, 1, BN, 1), lambda b, i: (b, i, 0, 0)),
        pl.BlockSpec((1, 1, BN, 1), lambda b, i: (b, i, 0, 0)),
        pl.BlockSpec((1, 1, M), lambda b, i: (b, 0, 0)),
        pl.BlockSpec((1, 1, M), lambda b, i: (b, 0, 0)),
        pl.BlockSpec((1, 1, M), lambda b, i: (b, 0, 0)),
    ]
    scratch_shapes = [
        pltpu.VMEM((M, C), f32),
        pltpu.VMEM((1, M), f32),
        pltpu.VMEM((1, M), i32),
        pltpu.VMEM((1, M), f32),
    ]
    sim, n0, v1r, v2r, n1, v1c, v2c = pl.pallas_call(
        functools.partial(_tc_body, BN, M, NI),
        grid=grid,
        in_specs=in_specs,
        out_specs=out_specs,
        out_shape=out_shape,
        scratch_shapes=scratch_shapes,
        interpret=interpret,
    )(desc0, desc1)
    return (sim,
            n0.reshape(B, N), v1r.reshape(B, N), v2r.reshape(B, N),
            n1.reshape(B, M), v1c.reshape(B, M), v2c.reshape(B, M))


def _sc_direction(base, CH, K, other_n, other_kpts_flat, idx_chunk,
                  v1_chunk, v2_chunk, m_out, s_out, mk_out_flat):
    rt2 = jnp.float32(RATIO_THRESH * RATIO_THRESH)
    dt2 = jnp.float32(DIST_THRESH * DIST_THRESH)
    lanes = jax.lax.iota(jnp.int32, 16)

    def step(j, _):
        off = j * 16
        idx = idx_chunk[pl.ds(off, 16)]
        back = plsc.load_gather(other_n, [idx])
        mutual = back == (base + off + lanes)
        v1 = v1_chunk[pl.ds(off, 16)]
        v2 = v2_chunk[pl.ds(off, 16)]
        d1 = 2.0 * (1.0 - v1)
        d2 = 2.0 * (1.0 - v2)
        valid = mutual & (d1 <= rt2 * d2) & (d1 <= dt2)
        m_out[pl.ds(off, 16)] = jnp.where(valid, idx, -1)
        s_out[pl.ds(off, 16)] = jnp.where(valid, (v1 + 1.0) * 0.5, 0.0)
        safe3 = jnp.clip(idx, 0, K - 1) * 3
        vf = valid.astype(jnp.float32)
        row3 = (off + lanes) * 3
        for c in range(3):
            comp = plsc.load_gather(other_kpts_flat, [safe3 + c]) * vf
            plsc.store_scatter(mk_out_flat, [row3 + c], comp)
        return 0

    jax.lax.fori_loop(0, CH // 16, step, 0)


def _sc_postprocess(near0, near1, v1r, v2r, v1c, v2c, kpts0, kpts1):
    B, N = near0.shape
    M = near1.shape[1]
    f32, i32 = jnp.float32, jnp.int32
    info = plsc.get_sparse_core_info()
    NW = info.num_cores * info.num_subcores
    CH0 = (B * N) // NW
    CH1 = (B * M) // NW
    W0 = N // CH0
    W1 = M // CH1
    mesh = plsc.VectorSubcoreMesh(core_axis_name="c", subcore_axis_name="s")

    @functools.partial(
        pl.kernel, mesh=mesh,
        compiler_params=pltpu.CompilerParams(needs_layout_passes=False),
        out_type=[
            jax.ShapeDtypeStruct((B, N), i32),
            jax.ShapeDtypeStruct((B, M), i32),
            jax.ShapeDtypeStruct((B, N), f32),
            jax.ShapeDtypeStruct((B, M), f32),
            jax.ShapeDtypeStruct((B, M * 3), f32),
            jax.ShapeDtypeStruct((B, N * 3), f32),
        ],
        scratch_types=[
            pltpu.VMEM((M,), i32),
            pltpu.VMEM((M * 3,), f32),
            pltpu.VMEM((N,), i32),
            pltpu.VMEM((N * 3,), f32),
            pltpu.VMEM((CH0,), i32),
            pltpu.VMEM((CH0,), f32),
            pltpu.VMEM((CH0,), f32),
            pltpu.VMEM((CH0,), i32),
            pltpu.VMEM((CH0,), f32),
            pltpu.VMEM((CH0 * 3,), f32),
            pltpu.VMEM((CH1,), i32),
            pltpu.VMEM((CH1,), f32),
            pltpu.VMEM((CH1,), f32),
            pltpu.VMEM((CH1,), i32),
            pltpu.VMEM((CH1,), f32),
            pltpu.VMEM((CH1 * 3,), f32),
        ],
    )
    def post(n0_hbm, n1_hbm, v1r_hbm, v2r_hbm, v1c_hbm, v2c_hbm,
             k0_hbm, k1_hbm,
             m0_hbm, m1_hbm, s0_hbm, s1_hbm, mk0_hbm, mk1_hbm,
             n1_v, k1_v, n0_v, k0_v,
             i0_v, a0_v, b0_v, mm0_v, ss0_v, kk0_v,
             i1_v, a1_v, b1_v, mm1_v, ss1_v, kk1_v):
        wid = jax.lax.axis_index("s") * info.num_cores + jax.lax.axis_index("c")

        b = wid // W0
        base = (wid % W0) * CH0
        pltpu.sync_copy(n1_hbm.at[b], n1_v)
        pltpu.sync_copy(k1_hbm.at[b], k1_v)
        pltpu.sync_copy(n0_hbm.at[b, pl.ds(base, CH0)], i0_v)
        pltpu.sync_copy(v1r_hbm.at[b, pl.ds(base, CH0)], a0_v)
        pltpu.sync_copy(v2r_hbm.at[b, pl.ds(base, CH0)], b0_v)
        _sc_direction(base, CH0, M, n1_v, k1_v, i0_v, a0_v, b0_v,
                      mm0_v, ss0_v, kk0_v)
        pltpu.sync_copy(mm0_v, m0_hbm.at[b, pl.ds(base, CH0)])
        pltpu.sync_copy(ss0_v, s0_hbm.at[b, pl.ds(base, CH0)])
        pltpu.sync_copy(kk0_v, mk1_hbm.at[b, pl.ds(base * 3, CH0 * 3)])

        b1 = wid // W1
        base1 = (wid % W1) * CH1
        pltpu.sync_copy(n0_hbm.at[b1], n0_v)
        pltpu.sync_copy(k0_hbm.at[b1], k0_v)
        pltpu.sync_copy(n1_hbm.at[b1, pl.ds(base1, CH1)], i1_v)
        pltpu.sync_copy(v1c_hbm.at[b1, pl.ds(base1, CH1)], a1_v)
        pltpu.sync_copy(v2c_hbm.at[b1, pl.ds(base1, CH1)], b1_v)
        _sc_direction(base1, CH1, N, n0_v, k0_v, i1_v, a1_v, b1_v,
                      mm1_v, ss1_v, kk1_v)
        pltpu.sync_copy(mm1_v, m1_hbm.at[b1, pl.ds(base1, CH1)])
        pltpu.sync_copy(ss1_v, s1_hbm.at[b1, pl.ds(base1, CH1)])
        pltpu.sync_copy(kk1_v, mk0_hbm.at[b1, pl.ds(base1 * 3, CH1 * 3)])

    m0, m1, s0, s1, mk0f, mk1f = post(
        near0, near1, v1r, v2r, v1c, v2c,
        kpts0.reshape(B, N * 3), kpts1.reshape(B, M * 3))
    return m0, m1, s0, s1, mk0f.reshape(B, M, 3), mk1f.reshape(B, N, 3)


def kernel(desc0, desc1, kpts0, kpts1):
    sim, n0, v1r, v2r, n1, v1c, v2c = _tc_matmul_top2(desc0, desc1)
    m0, m1, s0, s1, mk0, mk1 = _sc_postprocess(
        n0, n1, v1r, v2r, v1c, v2c, kpts0, kpts1)
    return (m0, m1, s0, s1, mk0, mk1, sim)


# speedup vs baseline: 1.1709x; 1.1709x over previous
import functools

import jax
import jax.numpy as jnp
from jax.experimental import pallas as pl
from jax.experimental.pallas import tpu as pltpu
from jax.experimental.pallas import tpu_sc as plsc

RATIO_THRESH = 0.95
DIST_THRESH = 1.0
NEG = -3.0


def _tc_body(BN, M, NI, d0_ref, d1_ref, sim_ref, n0_ref, v1r_ref, v2r_ref,
             n1_ref, v1c_ref, v2c_ref, d1n_ref, v1s, i1s, v2s):
    i = pl.program_id(1)

    @pl.when(i == 0)
    def _():
        d1 = d1_ref[0]
        nrm = jnp.sqrt(jnp.sum(d1 * d1, axis=1, keepdims=True))
        d1n_ref[...] = d1 / jnp.maximum(nrm, 1e-12)

    d0 = d0_ref[0]
    nrm0 = jnp.sqrt(jnp.sum(d0 * d0, axis=1, keepdims=True))
    d0n = d0 / jnp.maximum(nrm0, 1e-12)
    sim = jax.lax.dot_general(
        d0n, d1n_ref[...],
        dimension_numbers=(((1,), (1,)), ((), ())),
        preferred_element_type=jnp.float32)
    sim_ref[0] = sim

    colf = jax.lax.broadcasted_iota(jnp.int32, (M, 1), 0).astype(jnp.float32)
    v1 = jnp.max(sim, axis=1, keepdims=True)
    eq = sim == v1
    eqf = eq.astype(jnp.float32)
    i1 = jax.lax.dot_general(
        eqf, colf, dimension_numbers=(((1,), (0,)), ((), ())),
        preferred_element_type=jnp.float32).astype(jnp.int32)
    v2 = jnp.max(jnp.where(eq, NEG, sim), axis=1, keepdims=True)
    n0_ref[0, 0] = i1
    v1r_ref[0, 0] = v1
    v2r_ref[0, 0] = v2

    rowf = (jax.lax.broadcasted_iota(jnp.int32, (1, BN), 1) + i * BN
            ).astype(jnp.float32)
    c1 = jnp.max(sim, axis=0, keepdims=True)
    eqc = sim == c1
    eqcf = eqc.astype(jnp.float32)
    gi = jax.lax.dot_general(
        rowf, eqcf, dimension_numbers=(((1,), (0,)), ((), ())),
        preferred_element_type=jnp.float32).astype(jnp.int32)
    c2 = jnp.max(jnp.where(eqc, NEG, sim), axis=0, keepdims=True)

    @pl.when(i == 0)
    def _():
        v1s[...] = c1
        i1s[...] = gi
        v2s[...] = c2

    @pl.when(i > 0)
    def _():
        pv1, pi1, pv2 = v1s[...], i1s[...], v2s[...]
        keep = pv1 >= c1
        v2s[...] = jnp.where(keep, jnp.maximum(pv2, c1), jnp.maximum(pv1, c2))
        i1s[...] = jnp.where(keep, pi1, gi)
        v1s[...] = jnp.maximum(pv1, c1)

    @pl.when(i == NI - 1)
    def _():
        n1_ref[0] = i1s[...]
        v1c_ref[0] = v1s[...]
        v2c_ref[0] = v2s[...]


def _tc_matmul_top2(desc0, desc1, BN=256, interpret=False):
    B, N, C = desc0.shape
    M = desc1.shape[1]
    NI = N // BN
    f32, i32 = jnp.float32, jnp.int32
    out_shape = [
        jax.ShapeDtypeStruct((B, N, M), f32),
        jax.ShapeDtypeStruct((B, NI, BN, 1), i32),
        jax.ShapeDtypeStruct((B, NI, BN, 1), f32),
        jax.ShapeDtypeStruct((B, NI, BN, 1), f32),
        jax.ShapeDtypeStruct((B, 1, M), i32),
        jax.ShapeDtypeStruct((B, 1, M), f32),
        jax.ShapeDtypeStruct((B, 1, M), f32),
    ]
    grid = (B, NI)
    in_specs = [
        pl.BlockSpec((1, BN, C), lambda b, i: (b, i, 0)),
        pl.BlockSpec((1, M, C), lambda b, i: (b, 0, 0)),
    ]
    out_specs = [
        pl.BlockSpec((1, BN, M), lambda b, i: (b, i, 0)),
        pl.BlockSpec((1, 1, BN, 1), lambda b, i: (b, i, 0, 0)),
        pl.BlockSpec((1, 1, BN, 1), lambda b, i: (b, i, 0, 0)),
        pl.BlockSpec((1, 1, BN, 1), lambda b, i: (b, i, 0, 0)),
        pl.BlockSpec((1, 1, M), lambda b, i: (b, 0, 0)),
        pl.BlockSpec((1, 1, M), lambda b, i: (b, 0, 0)),
        pl.BlockSpec((1, 1, M), lambda b, i: (b, 0, 0)),
    ]
    scratch_shapes = [
        pltpu.VMEM((M, C), f32),
        pltpu.VMEM((1, M), f32),
        pltpu.VMEM((1, M), i32),
        pltpu.VMEM((1, M), f32),
    ]
    sim, n0, v1r, v2r, n1, v1c, v2c = pl.pallas_call(
        functools.partial(_tc_body, BN, M, NI),
        grid=grid,
        in_specs=in_specs,
        out_specs=out_specs,
        out_shape=out_shape,
        scratch_shapes=scratch_shapes,
        interpret=interpret,
    )(desc0, desc1)
    return (sim,
            n0.reshape(B, N), v1r.reshape(B, N), v2r.reshape(B, N),
            n1.reshape(B, M), v1c.reshape(B, M), v2c.reshape(B, M))


def _sc_direction(base, CH, K, other_n, other_kpts_flat, idx_chunk,
                  v1_chunk, v2_chunk, m_out, s_out, mk_out_flat):
    rt2 = jnp.float32(RATIO_THRESH * RATIO_THRESH)
    dt2 = jnp.float32(DIST_THRESH * DIST_THRESH)
    lanes = jax.lax.iota(jnp.int32, 16)

    def step(j, _):
        off = j * 16
        idx = idx_chunk[pl.ds(off, 16)]
        back = plsc.load_gather(other_n, [idx])
        mutual = back == (base + off + lanes)
        v1 = v1_chunk[pl.ds(off, 16)]
        v2 = v2_chunk[pl.ds(off, 16)]
        d1 = 2.0 * (1.0 - v1)
        d2 = 2.0 * (1.0 - v2)
        valid = mutual & (d1 <= rt2 * d2) & (d1 <= dt2)
        m_out[pl.ds(off, 16)] = jnp.where(valid, idx, -1)
        s_out[pl.ds(off, 16)] = jnp.where(valid, (v1 + 1.0) * 0.5, 0.0)
        safe3 = jnp.clip(idx, 0, K - 1) * 3
        vf = valid.astype(jnp.float32)
        row3 = (off + lanes) * 3
        for c in range(3):
            comp = plsc.load_gather(other_kpts_flat, [safe3 + c]) * vf
            plsc.store_scatter(mk_out_flat, [row3 + c], comp)
        return 0

    jax.lax.fori_loop(0, CH // 16, step, 0)


def _sc_postprocess(near0, near1, v1r, v2r, v1c, v2c, kpts0, kpts1):
    B, N = near0.shape
    M = near1.shape[1]
    f32, i32 = jnp.float32, jnp.int32
    info = plsc.get_sparse_core_info()
    NW = info.num_cores * info.num_subcores
    CH0 = (B * N) // NW
    CH1 = (B * M) // NW
    W0 = N // CH0
    W1 = M // CH1
    mesh = plsc.VectorSubcoreMesh(core_axis_name="c", subcore_axis_name="s")

    @functools.partial(
        pl.kernel, mesh=mesh,
        compiler_params=pltpu.CompilerParams(needs_layout_passes=False),
        out_type=[
            jax.ShapeDtypeStruct((B, N), i32),
            jax.ShapeDtypeStruct((B, M), i32),
            jax.ShapeDtypeStruct((B, N), f32),
            jax.ShapeDtypeStruct((B, M), f32),
            jax.ShapeDtypeStruct((B, M * 3), f32),
            jax.ShapeDtypeStruct((B, N * 3), f32),
        ],
        scratch_types=[
            pltpu.VMEM((M,), i32),
            pltpu.VMEM((M * 3,), f32),
            pltpu.VMEM((N,), i32),
            pltpu.VMEM((N * 3,), f32),
            pltpu.VMEM((CH0,), i32),
            pltpu.VMEM((CH0,), f32),
            pltpu.VMEM((CH0,), f32),
            pltpu.VMEM((CH0,), i32),
            pltpu.VMEM((CH0,), f32),
            pltpu.VMEM((CH0 * 3,), f32),
            pltpu.VMEM((CH1,), i32),
            pltpu.VMEM((CH1,), f32),
            pltpu.VMEM((CH1,), f32),
            pltpu.VMEM((CH1,), i32),
            pltpu.VMEM((CH1,), f32),
            pltpu.VMEM((CH1 * 3,), f32),
            pltpu.SemaphoreType.DMA,
            pltpu.SemaphoreType.DMA,
        ],
    )
    def post(n0_hbm, n1_hbm, v1r_hbm, v2r_hbm, v1c_hbm, v2c_hbm,
             k0_hbm, k1_hbm,
             m0_hbm, m1_hbm, s0_hbm, s1_hbm, mk0_hbm, mk1_hbm,
             n1_v, k1_v, n0_v, k0_v,
             i0_v, a0_v, b0_v, mm0_v, ss0_v, kk0_v,
             i1_v, a1_v, b1_v, mm1_v, ss1_v, kk1_v,
             sem_in, sem_out):
        wid = jax.lax.axis_index("s") * info.num_cores + jax.lax.axis_index("c")

        b = wid // W0
        base = (wid % W0) * CH0
        b1 = wid // W1
        base1 = (wid % W1) * CH1

        ins = [
            pltpu.async_copy(n1_hbm.at[b], n1_v, sem_in),
            pltpu.async_copy(k1_hbm.at[b], k1_v, sem_in),
            pltpu.async_copy(n0_hbm.at[b, pl.ds(base, CH0)], i0_v, sem_in),
            pltpu.async_copy(v1r_hbm.at[b, pl.ds(base, CH0)], a0_v, sem_in),
            pltpu.async_copy(v2r_hbm.at[b, pl.ds(base, CH0)], b0_v, sem_in),
            pltpu.async_copy(n0_hbm.at[b1], n0_v, sem_in),
            pltpu.async_copy(k0_hbm.at[b1], k0_v, sem_in),
            pltpu.async_copy(n1_hbm.at[b1, pl.ds(base1, CH1)], i1_v, sem_in),
            pltpu.async_copy(v1c_hbm.at[b1, pl.ds(base1, CH1)], a1_v, sem_in),
            pltpu.async_copy(v2c_hbm.at[b1, pl.ds(base1, CH1)], b1_v, sem_in),
        ]
        for c in ins:
            c.wait()

        _sc_direction(base, CH0, M, n1_v, k1_v, i0_v, a0_v, b0_v,
                      mm0_v, ss0_v, kk0_v)
        outs = [
            pltpu.async_copy(mm0_v, m0_hbm.at[b, pl.ds(base, CH0)], sem_out),
            pltpu.async_copy(ss0_v, s0_hbm.at[b, pl.ds(base, CH0)], sem_out),
            pltpu.async_copy(
                kk0_v, mk1_hbm.at[b, pl.ds(base * 3, CH0 * 3)], sem_out),
        ]

        _sc_direction(base1, CH1, N, n0_v, k0_v, i1_v, a1_v, b1_v,
                      mm1_v, ss1_v, kk1_v)
        outs += [
            pltpu.async_copy(mm1_v, m1_hbm.at[b1, pl.ds(base1, CH1)], sem_out),
            pltpu.async_copy(ss1_v, s1_hbm.at[b1, pl.ds(base1, CH1)], sem_out),
            pltpu.async_copy(
                kk1_v, mk0_hbm.at[b1, pl.ds(base1 * 3, CH1 * 3)], sem_out),
        ]
        for c in outs:
            c.wait()

    m0, m1, s0, s1, mk0f, mk1f = post(
        near0, near1, v1r, v2r, v1c, v2c,
        kpts0.reshape(B, N * 3), kpts1.reshape(B, M * 3))
    return m0, m1, s0, s1, mk0f.reshape(B, M, 3), mk1f.reshape(B, N, 3)


def kernel(desc0, desc1, kpts0, kpts1):
    sim, n0, v1r, v2r, n1, v1c, v2c = _tc_matmul_top2(desc0, desc1)
    m0, m1, s0, s1, mk0, mk1 = _sc_postprocess(
        n0, n1, v1r, v2r, v1c, v2c, kpts0, kpts1)
    return (m0, m1, s0, s1, mk0, mk1, sim)


# speedup vs baseline: 1.2329x; 1.0530x over previous
import functools

import jax
import jax.numpy as jnp
from jax.experimental import pallas as pl
from jax.experimental.pallas import tpu as pltpu
from jax.experimental.pallas import tpu_sc as plsc

RATIO_THRESH = 0.95
DIST_THRESH = 1.0
NEG = -3.0


def _tc_body(BN, M, NI, d0_ref, d1_ref, sim_ref, n0_ref, v1r_ref, v2r_ref,
             n1_ref, v1c_ref, v2c_ref, d1n_ref, v1s, i1s, v2s):
    i = pl.program_id(1)

    @pl.when(i == 0)
    def _():
        d1 = d1_ref[0]
        nrm = jnp.sqrt(jnp.sum(d1 * d1, axis=1, keepdims=True))
        d1n_ref[...] = d1 / jnp.maximum(nrm, 1e-12)

    d0 = d0_ref[0]
    nrm0 = jnp.sqrt(jnp.sum(d0 * d0, axis=1, keepdims=True))
    d0n = d0 / jnp.maximum(nrm0, 1e-12)
    sim = jax.lax.dot_general(
        d0n, d1n_ref[...],
        dimension_numbers=(((1,), (1,)), ((), ())),
        preferred_element_type=jnp.float32)
    sim_ref[0] = sim

    coli = jax.lax.broadcasted_iota(jnp.int32, (M, 2), 0)
    sel = jax.lax.broadcasted_iota(jnp.int32, (M, 2), 1) == 0
    colhl = jnp.where(sel, coli // 8, coli % 8).astype(jnp.bfloat16)
    v1 = jnp.max(sim, axis=1, keepdims=True)
    eq = sim == v1
    hl = jax.lax.dot_general(
        eq.astype(jnp.bfloat16), colhl,
        dimension_numbers=(((1,), (0,)), ((), ())),
        preferred_element_type=jnp.float32).astype(jnp.int32)
    i1 = hl[:, 0:1] * 8 + hl[:, 1:2]
    v2 = jnp.max(jnp.where(eq, NEG, sim), axis=1, keepdims=True)
    n0_ref[0, 0] = i1
    v1r_ref[0, 0] = v1
    v2r_ref[0, 0] = v2

    rowf = (jax.lax.broadcasted_iota(jnp.int32, (1, BN), 1) + i * BN
            ).astype(jnp.float32)
    c1 = jnp.max(sim, axis=0, keepdims=True)
    eqc = sim == c1
    eqcf = eqc.astype(jnp.float32)
    gi = jax.lax.dot_general(
        rowf, eqcf, dimension_numbers=(((1,), (0,)), ((), ())),
        preferred_element_type=jnp.float32).astype(jnp.int32)
    c2 = jnp.max(jnp.where(eqc, NEG, sim), axis=0, keepdims=True)

    @pl.when(i == 0)
    def _():
        v1s[...] = c1
        i1s[...] = gi
        v2s[...] = c2

    @pl.when(i > 0)
    def _():
        pv1, pi1, pv2 = v1s[...], i1s[...], v2s[...]
        keep = pv1 >= c1
        v2s[...] = jnp.where(keep, jnp.maximum(pv2, c1), jnp.maximum(pv1, c2))
        i1s[...] = jnp.where(keep, pi1, gi)
        v1s[...] = jnp.maximum(pv1, c1)

    @pl.when(i == NI - 1)
    def _():
        n1_ref[0] = i1s[...]
        v1c_ref[0] = v1s[...]
        v2c_ref[0] = v2s[...]


def _tc_matmul_top2(desc0, desc1, BN=256, interpret=False):
    B, N, C = desc0.shape
    M = desc1.shape[1]
    NI = N // BN
    f32, i32 = jnp.float32, jnp.int32
    out_shape = [
        jax.ShapeDtypeStruct((B, N, M), f32),
        jax.ShapeDtypeStruct((B, NI, BN, 1), i32),
        jax.ShapeDtypeStruct((B, NI, BN, 1), f32),
        jax.ShapeDtypeStruct((B, NI, BN, 1), f32),
        jax.ShapeDtypeStruct((B, 1, M), i32),
        jax.ShapeDtypeStruct((B, 1, M), f32),
        jax.ShapeDtypeStruct((B, 1, M), f32),
    ]
    grid = (B, NI)
    in_specs = [
        pl.BlockSpec((1, BN, C), lambda b, i: (b, i, 0)),
        pl.BlockSpec((1, M, C), lambda b, i: (b, 0, 0)),
    ]
    out_specs = [
        pl.BlockSpec((1, BN, M), lambda b, i: (b, i, 0)),
        pl.BlockSpec((1, 1, BN, 1), lambda b, i: (b, i, 0, 0)),
        pl.BlockSpec((1, 1, BN, 1), lambda b, i: (b, i, 0, 0)),
        pl.BlockSpec((1, 1, BN, 1), lambda b, i: (b, i, 0, 0)),
        pl.BlockSpec((1, 1, M), lambda b, i: (b, 0, 0)),
        pl.BlockSpec((1, 1, M), lambda b, i: (b, 0, 0)),
        pl.BlockSpec((1, 1, M), lambda b, i: (b, 0, 0)),
    ]
    scratch_shapes = [
        pltpu.VMEM((M, C), f32),
        pltpu.VMEM((1, M), f32),
        pltpu.VMEM((1, M), i32),
        pltpu.VMEM((1, M), f32),
    ]
    sim, n0, v1r, v2r, n1, v1c, v2c = pl.pallas_call(
        functools.partial(_tc_body, BN, M, NI),
        grid=grid,
        in_specs=in_specs,
        out_specs=out_specs,
        out_shape=out_shape,
        scratch_shapes=scratch_shapes,
        interpret=interpret,
    )(desc0, desc1)
    return (sim,
            n0.reshape(B, N), v1r.reshape(B, N), v2r.reshape(B, N),
            n1.reshape(B, M), v1c.reshape(B, M), v2c.reshape(B, M))


def _sc_direction(base, CH, K, other_n, other_kpts_flat, idx_chunk,
                  v1_chunk, v2_chunk, m_out, s_out, mk_out_flat):
    rt2 = jnp.float32(RATIO_THRESH * RATIO_THRESH)
    dt2 = jnp.float32(DIST_THRESH * DIST_THRESH)
    lanes = jax.lax.iota(jnp.int32, 16)

    def step(j, _):
        off = j * 16
        idx = idx_chunk[pl.ds(off, 16)]
        back = plsc.load_gather(other_n, [idx])
        mutual = back == (base + off + lanes)
        v1 = v1_chunk[pl.ds(off, 16)]
        v2 = v2_chunk[pl.ds(off, 16)]
        d1 = 2.0 * (1.0 - v1)
        d2 = 2.0 * (1.0 - v2)
        valid = mutual & (d1 <= rt2 * d2) & (d1 <= dt2)
        m_out[pl.ds(off, 16)] = jnp.where(valid, idx, -1)
        s_out[pl.ds(off, 16)] = jnp.where(valid, (v1 + 1.0) * 0.5, 0.0)
        safe3 = jnp.clip(idx, 0, K - 1) * 3
        vf = valid.astype(jnp.float32)
        row3 = (off + lanes) * 3
        for c in range(3):
            comp = plsc.load_gather(other_kpts_flat, [safe3 + c]) * vf
            plsc.store_scatter(mk_out_flat, [row3 + c], comp)
        return 0

    jax.lax.fori_loop(0, CH // 16, step, 0)


def _sc_postprocess(near0, near1, v1r, v2r, v1c, v2c, kpts0, kpts1):
    B, N = near0.shape
    M = near1.shape[1]
    f32, i32 = jnp.float32, jnp.int32
    info = plsc.get_sparse_core_info()
    NW = info.num_cores * info.num_subcores
    CH0 = (B * N) // NW
    CH1 = (B * M) // NW
    W0 = N // CH0
    W1 = M // CH1
    mesh = plsc.VectorSubcoreMesh(core_axis_name="c", subcore_axis_name="s")

    @functools.partial(
        pl.kernel, mesh=mesh,
        compiler_params=pltpu.CompilerParams(needs_layout_passes=False),
        out_type=[
            jax.ShapeDtypeStruct((B, N), i32),
            jax.ShapeDtypeStruct((B, M), i32),
            jax.ShapeDtypeStruct((B, N), f32),
            jax.ShapeDtypeStruct((B, M), f32),
            jax.ShapeDtypeStruct((B, M * 3), f32),
            jax.ShapeDtypeStruct((B, N * 3), f32),
        ],
        scratch_types=[
            pltpu.VMEM((M,), i32),
            pltpu.VMEM((M * 3,), f32),
            pltpu.VMEM((N,), i32),
            pltpu.VMEM((N * 3,), f32),
            pltpu.VMEM((CH0,), i32),
            pltpu.VMEM((CH0,), f32),
            pltpu.VMEM((CH0,), f32),
            pltpu.VMEM((CH0,), i32),
            pltpu.VMEM((CH0,), f32),
            pltpu.VMEM((CH0 * 3,), f32),
            pltpu.VMEM((CH1,), i32),
            pltpu.VMEM((CH1,), f32),
            pltpu.VMEM((CH1,), f32),
            pltpu.VMEM((CH1,), i32),
            pltpu.VMEM((CH1,), f32),
            pltpu.VMEM((CH1 * 3,), f32),
            pltpu.SemaphoreType.DMA,
            pltpu.SemaphoreType.DMA,
        ],
    )
    def post(n0_hbm, n1_hbm, v1r_hbm, v2r_hbm, v1c_hbm, v2c_hbm,
             k0_hbm, k1_hbm,
             m0_hbm, m1_hbm, s0_hbm, s1_hbm, mk0_hbm, mk1_hbm,
             n1_v, k1_v, n0_v, k0_v,
             i0_v, a0_v, b0_v, mm0_v, ss0_v, kk0_v,
             i1_v, a1_v, b1_v, mm1_v, ss1_v, kk1_v,
             sem_in, sem_out):
        wid = jax.lax.axis_index("s") * info.num_cores + jax.lax.axis_index("c")

        b = wid // W0
        base = (wid % W0) * CH0
        b1 = wid // W1
        base1 = (wid % W1) * CH1

        ins = [
            pltpu.async_copy(n1_hbm.at[b], n1_v, sem_in),
            pltpu.async_copy(k1_hbm.at[b], k1_v, sem_in),
            pltpu.async_copy(n0_hbm.at[b, pl.ds(base, CH0)], i0_v, sem_in),
            pltpu.async_copy(v1r_hbm.at[b, pl.ds(base, CH0)], a0_v, sem_in),
            pltpu.async_copy(v2r_hbm.at[b, pl.ds(base, CH0)], b0_v, sem_in),
            pltpu.async_copy(n0_hbm.at[b1], n0_v, sem_in),
            pltpu.async_copy(k0_hbm.at[b1], k0_v, sem_in),
            pltpu.async_copy(n1_hbm.at[b1, pl.ds(base1, CH1)], i1_v, sem_in),
            pltpu.async_copy(v1c_hbm.at[b1, pl.ds(base1, CH1)], a1_v, sem_in),
            pltpu.async_copy(v2c_hbm.at[b1, pl.ds(base1, CH1)], b1_v, sem_in),
        ]
        for c in ins:
            c.wait()

        _sc_direction(base, CH0, M, n1_v, k1_v, i0_v, a0_v, b0_v,
                      mm0_v, ss0_v, kk0_v)
        outs = [
            pltpu.async_copy(mm0_v, m0_hbm.at[b, pl.ds(base, CH0)], sem_out),
            pltpu.async_copy(ss0_v, s0_hbm.at[b, pl.ds(base, CH0)], sem_out),
            pltpu.async_copy(
                kk0_v, mk1_hbm.at[b, pl.ds(base * 3, CH0 * 3)], sem_out),
        ]

        _sc_direction(base1, CH1, N, n0_v, k0_v, i1_v, a1_v, b1_v,
                      mm1_v, ss1_v, kk1_v)
        outs += [
            pltpu.async_copy(mm1_v, m1_hbm.at[b1, pl.ds(base1, CH1)], sem_out),
            pltpu.async_copy(ss1_v, s1_hbm.at[b1, pl.ds(base1, CH1)], sem_out),
            pltpu.async_copy(
                kk1_v, mk0_hbm.at[b1, pl.ds(base1 * 3, CH1 * 3)], sem_out),
        ]
        for c in outs:
            c.wait()

    m0, m1, s0, s1, mk0f, mk1f = post(
        near0, near1, v1r, v2r, v1c, v2c,
        kpts0.reshape(B, N * 3), kpts1.reshape(B, M * 3))
    return m0, m1, s0, s1, mk0f.reshape(B, M, 3), mk1f.reshape(B, N, 3)


def kernel(desc0, desc1, kpts0, kpts1):
    sim, n0, v1r, v2r, n1, v1c, v2c = _tc_matmul_top2(desc0, desc1)
    m0, m1, s0, s1, mk0, mk1 = _sc_postprocess(
        n0, n1, v1r, v2r, v1c, v2c, kpts0, kpts1)
    return (m0, m1, s0, s1, mk0, mk1, sim)
